# Initial kernel scaffold; baseline (speedup 1.0000x reference)
import jax
import jax.numpy as jnp
from jax.experimental import pallas as pl


def kernel(pred, label):
    raise NotImplementedError("write your pallas kernel here")


# speedup vs baseline: 1.7297x; 1.7297x over previous
import jax
import jax.numpy as jnp
from jax import lax
from jax.experimental import pallas as pl
from jax.experimental.pallas import tpu as pltpu

_LAMBDA_POS = 0.5
_LAMBDA_RAD = 0.5
_LAMBDA_UNPAIR = 0.5
_N_PRED = 20
_M = 12


def _loss_body(pred_ref, label_ref, out_ref):
    pred = pred_ref[...]
    label = label_ref[...]
    px = pred[:, 0][None, :]
    py = pred[:, 1][None, :]
    pr = pred[:, 2][None, :]
    pp = pred[:, 3][None, :]
    lx = label[:, 0][:, None]
    ly = label[:, 1][:, None]
    lr = label[:, 2][:, None]
    dx = lx - px
    dy = ly - py
    dist = jnp.sqrt(dx * dx + dy * dy)
    cost = _LAMBDA_POS * dist + _LAMBDA_RAD * jnp.abs(lr - pr)
    amin = jnp.min(cost, axis=1, keepdims=True)
    col = lax.broadcasted_iota(jnp.int32, (_M, _N_PRED), 1)
    jidx = jnp.min(jnp.where(cost <= amin, col, _N_PRED), axis=1, keepdims=True)
    onehot = (col == jidx).astype(jnp.float32)
    sel_x = jnp.sum(onehot * px, axis=1)
    sel_y = jnp.sum(onehot * py, axis=1)
    sel_r = jnp.sum(onehot * pr, axis=1)
    sel_p = jnp.sum(onehot * pp, axis=1)
    lx1 = label[:, 0]
    ly1 = label[:, 1]
    lr1 = label[:, 2]
    ddx = lx1 - sel_x
    ddy = ly1 - sel_y
    pdist = jnp.sqrt(ddx * ddx + ddy * ddy)
    prob = -jnp.log(sel_p + 1e-6)
    loss_pair = jnp.sum(_LAMBDA_POS * pdist + _LAMBDA_RAD * jnp.abs(lr1 - sel_r) + prob)
    pair_mask = jnp.max(onehot, axis=0)
    unpair = (-jnp.log(1.0 - pp[0] + 1e-6) + _LAMBDA_RAD * pr[0]) * _LAMBDA_UNPAIR
    loss_unpair = jnp.sum(jnp.where(pair_mask == 0.0, unpair, 0.0))
    out_ref[0, 0] = loss_pair / _M + loss_unpair / (_N_PRED - _M)


def kernel(pred, label):
    out = pl.pallas_call(
        _loss_body,
        out_shape=jax.ShapeDtypeStruct((1, 1), jnp.float32),
        out_specs=pl.BlockSpec(memory_space=pltpu.SMEM),
    )(pred, label)
    return out[0, 0]
